# Initial kernel scaffold; baseline (speedup 1.0000x reference)
import jax
import jax.numpy as jnp
from jax.experimental import pallas as pl


def kernel(struct_x, struct_edge_index, struct_batch, func_x, func_edge_index, func_batch, params):
    raise NotImplementedError("write your pallas kernel here")


# speedup vs baseline: 4.0807x; 4.0807x over previous
import functools

import jax
import jax.numpy as jnp
from jax import lax
from jax.experimental import pallas as pl
from jax.experimental.pallas import tpu as pltpu
from jax.experimental.pallas import tpu_sc as plsc

N_NODES = 10000
N_EDGES = 320000
N_GRAPHS = 128
HID = 128
NTILE = 16
NCH = 160
CHUNK = 128
E_PAD = NTILE * NCH * CHUNK
ROWS_PER_CORE = 5120
AGG_ROWS = 5376
RPT = AGG_ROWS // NTILE
CPT = ROWS_PER_CORE // NTILE


def _sc_one(s, h_hbm, src_hbm, dst_hbm, out_hbm, base,
            srcbuf, dstbuf, stag0, stag1, aggsh, sem0, sem1):
    zv = jnp.zeros((16,), jnp.float32)

    def zrow(i, carry):
        for j in range(HID // 16):
            stag0[i, pl.ds(j * 16, 16)] = zv
        return carry
    lax.fori_loop(0, CHUNK, zrow, 0)
    pltpu.sync_copy(stag0, aggsh.at[pl.ds(s * RPT, CHUNK)])
    pltpu.sync_copy(stag0, aggsh.at[pl.ds(s * RPT + CHUNK, CHUNK)])
    pltpu.sync_copy(stag0.at[pl.ds(0, RPT - 2 * CHUNK)],
                    aggsh.at[pl.ds(s * RPT + 2 * CHUNK, RPT - 2 * CHUNK)])

    pltpu.sync_copy(src_hbm.at[s], srcbuf)
    pltpu.sync_copy(dst_hbm.at[s], dstbuf)

    plsc.subcore_barrier()

    pltpu.async_copy(h_hbm.at[srcbuf.at[0]], stag0, sem0)

    def step(i, carry):
        j = i * 2
        pltpu.async_copy(h_hbm.at[srcbuf.at[j + 1]], stag1, sem1)
        pltpu.make_async_copy(h_hbm.at[srcbuf.at[j]], stag0, sem0).wait()
        pltpu.sync_copy(stag0, aggsh.at[dstbuf.at[j]], add=True)

        @pl.when(j + 2 < NCH)
        def _():
            pltpu.async_copy(h_hbm.at[srcbuf.at[j + 2]], stag0, sem0)

        pltpu.make_async_copy(h_hbm.at[srcbuf.at[j + 1]], stag1, sem1).wait()
        pltpu.sync_copy(stag1, aggsh.at[dstbuf.at[j + 1]], add=True)
        return carry
    lax.fori_loop(0, NCH // 2, step, 0)

    plsc.subcore_barrier()

    if base == 0:
        pltpu.sync_copy(aggsh.at[pl.ds(s * CPT, CPT)],
                        out_hbm.at[pl.ds(s * CPT, CPT)])
    else:
        nlast = N_NODES - base - (NTILE - 1) * CPT

        @pl.when(s < NTILE - 1)
        def _():
            pltpu.sync_copy(aggsh.at[pl.ds(s * CPT, CPT)],
                            out_hbm.at[pl.ds(base + s * CPT, CPT)])

        @pl.when(s == NTILE - 1)
        def _():
            pltpu.sync_copy(aggsh.at[pl.ds((NTILE - 1) * CPT, nlast)],
                            out_hbm.at[pl.ds(base + (NTILE - 1) * CPT, nlast)])

    plsc.subcore_barrier()


def _seg_body(h, src_s, src_f, ds0, ds1, df0, df1, out_s, out_f,
              srcbuf, dstbuf, stag0, stag1, aggsh, sem0, sem1):
    c = lax.axis_index("c")
    s = lax.axis_index("s")

    scr = (srcbuf, dstbuf, stag0, stag1, aggsh, sem0, sem1)

    @pl.when(c == 0)
    def _():
        _sc_one(s, h.at[0], src_s, ds0, out_s, 0, *scr)
        _sc_one(s, h.at[1], src_f, df0, out_f, 0, *scr)

    @pl.when(c == 1)
    def _():
        _sc_one(s, h.at[0], src_s, ds1, out_s, ROWS_PER_CORE, *scr)
        _sc_one(s, h.at[1], src_f, df1, out_f, ROWS_PER_CORE, *scr)


@functools.cache
def _make_seg_call():
    return pl.kernel(
        _seg_body,
        out_type=[jax.ShapeDtypeStruct((N_NODES, HID), jnp.float32),
                  jax.ShapeDtypeStruct((N_NODES, HID), jnp.float32)],
        mesh=plsc.VectorSubcoreMesh(core_axis_name="c", subcore_axis_name="s"),
        scratch_types=[
            pltpu.VMEM((NCH, CHUNK), jnp.int32),
            pltpu.VMEM((NCH, CHUNK), jnp.int32),
            pltpu.VMEM((CHUNK, HID), jnp.float32),
            pltpu.VMEM((CHUNK, HID), jnp.float32),
            pltpu.VMEM_SHARED((AGG_ROWS, HID), jnp.float32),
            pltpu.SemaphoreType.DMA,
            pltpu.SemaphoreType.DMA,
        ],
    )


def _seg_call(h, src_s, src_f, ds0, ds1, df0, df1):
    return _make_seg_call()(h, src_s, src_f, ds0, ds1, df0, df1)


def _prep_edges(ei):
    src = ei[0].astype(jnp.int32)
    dst = ei[1].astype(jnp.int32)
    padn = E_PAD - N_EDGES
    ar = jnp.arange(padn, dtype=jnp.int32)
    pad_src = (ar * 97) % N_NODES
    pad_dst = N_NODES + (ar % 128)
    src_p = jnp.concatenate([src, pad_src])
    dst_p = jnp.concatenate([dst, pad_dst])
    sac = ROWS_PER_CORE + (dst_p % (AGG_ROWS - ROWS_PER_CORE))
    d0 = jnp.where(dst_p < ROWS_PER_CORE, dst_p, sac)
    d1 = jnp.where(dst_p >= ROWS_PER_CORE,
                   jnp.minimum(dst_p - ROWS_PER_CORE, AGG_ROWS - 1), sac)
    shp = (NTILE, NCH, CHUNK)
    return (src_p.reshape(shp), d0.reshape(shp), d1.reshape(shp))


ROW_BLK = 2000


def _mlp_body(scale_ref, h_ref, agg_ref, w1_ref, b1_ref, w2_ref, b2_ref,
              out_ref, *, last):
    x = scale_ref[0] * h_ref[0] + agg_ref[0]
    y = jnp.dot(x, w1_ref[0], preferred_element_type=jnp.float32) + b1_ref[0]
    y = jnp.maximum(y, 0.0)
    z = jnp.dot(y, w2_ref[0], preferred_element_type=jnp.float32) + b2_ref[0]
    out_ref[0] = z if last else jnp.maximum(z, 0.0)


def _mlp_call(scale, h, agg, w1, b1, w2, b2, last):
    grid = (2, N_NODES // ROW_BLK)
    bs_small = pl.BlockSpec((1, 1, HID), lambda b, r: (b, 0, 0))
    bs_w = pl.BlockSpec((1, HID, HID), lambda b, r: (b, 0, 0))
    bs_h = pl.BlockSpec((1, ROW_BLK, HID), lambda b, r: (b, r, 0))
    return pl.pallas_call(
        functools.partial(_mlp_body, last=last),
        grid=grid,
        in_specs=[bs_small, bs_h, bs_h, bs_w, bs_small, bs_w, bs_small],
        out_specs=bs_h,
        out_shape=jax.ShapeDtypeStruct((2, N_NODES, HID), jnp.float32),
    )(scale, h, agg, w1, b1, w2, b2)


def _heads_body(h_ref, m_ref,
                a1_ref, a1b_ref, a2_ref, a2b_ref, spp_ref,
                genw_ref, genb_ref, fpp_ref,
                g1a_ref, g1b_ref, g1bias_ref, g2_ref, g2b_ref,
                fw1a_ref, fw1b_ref, fb1_ref, fw2_ref, fb2_ref,
                clw_ref, clb_ref,
                logits_ref, ortho_ref):
    f32 = jnp.float32
    dn = (((0,), (0,)), ((), ()))

    def pool(m, hh):
        ssum = lax.dot_general(m, hh, dn, preferred_element_type=f32)
        cnt8 = lax.dot_general(m, jnp.ones((N_NODES, 8), f32), dn,
                               preferred_element_type=f32)
        cnt = cnt8[:, 0:1]
        return ssum / jnp.maximum(cnt, 1.0)

    sf = pool(m_ref[0], h_ref[0])
    ff = pool(m_ref[1], h_ref[1])

    z = jnp.maximum(jnp.dot(sf, a1_ref[...], preferred_element_type=f32)
                    + a1b_ref[...], 0.0)
    wl = jnp.dot(z, a2_ref[...], preferred_element_type=f32) + a2b_ref[...]
    wl = wl - jnp.max(wl, axis=-1, keepdims=True)
    we = jnp.exp(wl)
    w = we / jnp.sum(we, axis=-1, keepdims=True)
    sf = sf + jnp.dot(w, spp_ref[...], preferred_element_type=f32)

    dyn = jnp.dot(ff, genw_ref[...], preferred_element_type=f32) + genb_ref[...]
    static = jnp.sum(fpp_ref[...], axis=0, keepdims=True) / 5.0
    g = jnp.maximum(jnp.dot(ff, g1a_ref[...], preferred_element_type=f32)
                    + jnp.dot(ff, g1b_ref[...], preferred_element_type=f32)
                    + g1bias_ref[...], 0.0)
    z2 = jnp.dot(g, g2_ref[...], preferred_element_type=f32) + g2b_ref[...]
    gate = 1.0 / (1.0 + jnp.exp(-z2[:, 0:1]))
    ff = ff + gate * dyn + (1.0 - gate) * static

    n1 = jnp.sqrt(jnp.sum(sf * sf, axis=1, keepdims=True))
    n2 = jnp.sqrt(jnp.sum(ff * ff, axis=1, keepdims=True))
    f1 = sf / jnp.maximum(n1, 1e-12)
    f2 = ff / jnp.maximum(n2, 1e-12)
    cross = lax.dot_general(f1, f2, (((1,), (1,)), ((), ())),
                            preferred_element_type=f32)
    ortho_ref[...] = jnp.reshape(
        jnp.sum(jnp.abs(cross)) * (0.01 / (N_GRAPHS * N_GRAPHS)), (1, 1))

    fused = jnp.maximum(jnp.dot(sf, fw1a_ref[...], preferred_element_type=f32)
                        + jnp.dot(ff, fw1b_ref[...], preferred_element_type=f32)
                        + fb1_ref[...], 0.0)
    fused = jnp.dot(fused, fw2_ref[...], preferred_element_type=f32) + fb2_ref[...]
    logits_ref[...] = (jnp.dot(fused, clw_ref[...], preferred_element_type=f32)
                       + clb_ref[...])


def _heads_call(h, m, hp):
    return pl.pallas_call(
        _heads_body,
        out_shape=[jax.ShapeDtypeStruct((N_GRAPHS, HID), jnp.float32),
                   jax.ShapeDtypeStruct((1, 1), jnp.float32)],
    )(h, m, *hp)


def _pad_to(a, shape):
    pads = [(0, t - s) for s, t in zip(a.shape, shape)]
    return jnp.pad(a, pads)


def _prep_head_params(p):
    f32 = jnp.float32
    a1 = _pad_to(p["sp_A1"], (HID, HID))
    a1b = _pad_to(p["sp_a1"].reshape(1, -1), (1, HID))
    a2 = _pad_to(p["sp_A2"], (HID, HID))
    a2b = jnp.full((1, HID), -1e30, f32).at[0, :5].set(p["sp_a2"])
    spp = _pad_to(p["sp_prompts"], (HID, HID))
    genw = p["fp_gen_W"]
    genb = p["fp_gen_b"].reshape(1, HID)
    fpp = _pad_to(p["fp_prompts"], (8, HID))
    g1a = p["fp_G1"][:HID]
    g1b = p["fp_G1"][HID:]
    g1bias = p["fp_g1"].reshape(1, HID)
    g2 = _pad_to(p["fp_G2"], (HID, HID))
    g2b = _pad_to(p["fp_g2"].reshape(1, -1), (1, HID))
    fw1a = p["fu_W1"][:HID]
    fw1b = p["fu_W1"][HID:]
    fb1 = p["fu_b1"].reshape(1, HID)
    fw2 = p["fu_W2"]
    fb2 = p["fu_b2"].reshape(1, HID)
    clw = _pad_to(p["cl_W"], (HID, HID))
    clb = _pad_to(p["cl_b"].reshape(1, -1), (1, HID))
    return [a1, a1b, a2, a2b, spp, genw, genb, fpp, g1a, g1b, g1bias, g2, g2b,
            fw1a, fw1b, fb1, fw2, fb2, clw, clb]


def kernel(struct_x, struct_edge_index, struct_batch,
           func_x, func_edge_index, func_batch, params):
    src_s, ds0, ds1 = _prep_edges(struct_edge_index)
    src_f, df0, df1 = _prep_edges(func_edge_index)

    h = jnp.stack([struct_x, func_x])
    sl = params["struct_layers"]
    fl = params["func_layers"]
    for l in range(5):
        agg_s, agg_f = _seg_call(h, src_s, src_f, ds0, ds1, df0, df1)
        agg = jnp.stack([agg_s, agg_f])
        scale = jnp.stack([
            jnp.full((1, HID), 1.0 + sl[l]["eps"], jnp.float32),
            jnp.full((1, HID), 1.0 + fl[l]["eps"], jnp.float32),
        ])
        w1 = jnp.stack([sl[l]["W1"], fl[l]["W1"]])
        b1 = jnp.stack([sl[l]["b1"].reshape(1, HID), fl[l]["b1"].reshape(1, HID)])
        w2 = jnp.stack([sl[l]["W2"], fl[l]["W2"]])
        b2 = jnp.stack([sl[l]["b2"].reshape(1, HID), fl[l]["b2"].reshape(1, HID)])
        h = _mlp_call(scale, h, agg, w1, b1, w2, b2, last=(l == 4))

    gids = jnp.arange(N_GRAPHS, dtype=jnp.int32)
    m = jnp.stack([
        (struct_batch.astype(jnp.int32)[:, None] == gids[None, :]).astype(jnp.float32),
        (func_batch.astype(jnp.int32)[:, None] == gids[None, :]).astype(jnp.float32),
    ])
    logits_full, ortho = _heads_call(h, m, _prep_head_params(params))
    return (logits_full[:, :2], ortho.reshape(()))


# speedup vs baseline: 5.8691x; 1.4383x over previous
import functools

import jax
import jax.numpy as jnp
from jax import lax
from jax.experimental import pallas as pl
from jax.experimental.pallas import tpu as pltpu
from jax.experimental.pallas import tpu_sc as plsc

N_NODES = 10000
N_EDGES = 320000
N_GRAPHS = 128
HID = 128
NTILE = 16
NCH = 100
CHUNK = 100
NCH2 = 20
E_HALF = N_EDGES // 2
AGG_ROWS = N_NODES
RPT = AGG_ROWS // NTILE


def _sc_one(s, h_hbm, src_hbm, dst_hbm, out_hbm,
            srcbuf, dstbuf, stag0, stag1, aggsh, sem0, sem1):
    zv = jnp.zeros((16,), jnp.float32)

    def zrow(i, carry):
        for j in range(HID // 16):
            stag0[i, pl.ds(j * 16, 16)] = zv
        return carry
    lax.fori_loop(0, CHUNK, zrow, 0)
    for k in range(RPT // CHUNK):
        pltpu.sync_copy(stag0, aggsh.at[pl.ds(s * RPT + k * CHUNK, CHUNK)])
    rem = RPT % CHUNK
    pltpu.sync_copy(stag0.at[pl.ds(0, rem)],
                    aggsh.at[pl.ds(s * RPT + RPT - rem, rem)])

    plsc.subcore_barrier()

    src_t = src_hbm.at[s]
    dst_t = dst_hbm.at[s]

    def segment(g, carry):
        pltpu.sync_copy(src_t.at[g], srcbuf)
        pltpu.sync_copy(dst_t.at[g], dstbuf)
        pltpu.async_copy(h_hbm.at[srcbuf.at[0]], stag0, sem0)

        def step(i, c2):
            j = i * 2
            pltpu.async_copy(h_hbm.at[srcbuf.at[j + 1]], stag1, sem1)
            pltpu.make_async_copy(h_hbm.at[srcbuf.at[j]], stag0, sem0).wait()
            pltpu.sync_copy(stag0, aggsh.at[dstbuf.at[j]], add=True)

            @pl.when(j + 2 < NCH2)
            def _():
                pltpu.async_copy(h_hbm.at[srcbuf.at[j + 2]], stag0, sem0)

            pltpu.make_async_copy(h_hbm.at[srcbuf.at[j + 1]], stag1, sem1).wait()
            pltpu.sync_copy(stag1, aggsh.at[dstbuf.at[j + 1]], add=True)
            return c2
        lax.fori_loop(0, NCH2 // 2, step, 0)
        return carry
    lax.fori_loop(0, NCH // NCH2, segment, 0)

    plsc.subcore_barrier()

    @pl.when(s < NTILE - 1)
    def _():
        pltpu.sync_copy(aggsh.at[pl.ds(s * 624, 624)],
                        out_hbm.at[pl.ds(s * 624, 624)])

    @pl.when(s == NTILE - 1)
    def _():
        pltpu.sync_copy(aggsh.at[pl.ds(15 * 624, 640)],
                        out_hbm.at[pl.ds(15 * 624, 640)])

    plsc.subcore_barrier()


def _seg_body(h, es0, ed0, es1, ed1, fs0, fd0, fs1, fd1, out_s, out_f,
              srcbuf, dstbuf, stag0, stag1, aggsh, sem0, sem1):
    c = lax.axis_index("c")
    s = lax.axis_index("s")

    scr = (srcbuf, dstbuf, stag0, stag1, aggsh, sem0, sem1)

    @pl.when(c == 0)
    def _():
        _sc_one(s, h.at[0], es0, ed0, out_s.at[0], *scr)
        _sc_one(s, h.at[1], fs0, fd0, out_f.at[0], *scr)

    @pl.when(c == 1)
    def _():
        _sc_one(s, h.at[0], es1, ed1, out_s.at[1], *scr)
        _sc_one(s, h.at[1], fs1, fd1, out_f.at[1], *scr)


@functools.cache
def _make_seg_call():
    return pl.kernel(
        _seg_body,
        out_type=[jax.ShapeDtypeStruct((2, N_NODES, HID), jnp.float32),
                  jax.ShapeDtypeStruct((2, N_NODES, HID), jnp.float32)],
        mesh=plsc.VectorSubcoreMesh(core_axis_name="c", subcore_axis_name="s"),
        scratch_types=[
            pltpu.VMEM((NCH2, CHUNK), jnp.int32),
            pltpu.VMEM((NCH2, CHUNK), jnp.int32),
            pltpu.VMEM((CHUNK, HID), jnp.float32),
            pltpu.VMEM((CHUNK, HID), jnp.float32),
            pltpu.VMEM_SHARED((AGG_ROWS, HID), jnp.float32),
            pltpu.SemaphoreType.DMA,
            pltpu.SemaphoreType.DMA,
        ],
    )


def _seg_call(h, edges_s, edges_f):
    return _make_seg_call()(h, *edges_s, *edges_f)


def _prep_edges(ei):
    shp = (NTILE, NCH // NCH2, NCH2, CHUNK)
    src = ei[0].astype(jnp.int32)
    dst = ei[1].astype(jnp.int32)
    return (src[:E_HALF].reshape(shp), dst[:E_HALF].reshape(shp),
            src[E_HALF:].reshape(shp), dst[E_HALF:].reshape(shp))


ROW_BLK = 2000


def _mlp_body(scale_ref, h_ref, agg_ref, w1_ref, b1_ref, w2_ref, b2_ref,
              out_ref, *, last):
    x = scale_ref[0] * h_ref[0] + agg_ref[0, 0] + agg_ref[0, 1]
    y = jnp.dot(x, w1_ref[0], preferred_element_type=jnp.float32) + b1_ref[0]
    y = jnp.maximum(y, 0.0)
    z = jnp.dot(y, w2_ref[0], preferred_element_type=jnp.float32) + b2_ref[0]
    out_ref[0] = z if last else jnp.maximum(z, 0.0)


def _mlp_call(scale, h, agg, w1, b1, w2, b2, last):
    grid = (2, N_NODES // ROW_BLK)
    bs_small = pl.BlockSpec((1, 1, HID), lambda b, r: (b, 0, 0))
    bs_w = pl.BlockSpec((1, HID, HID), lambda b, r: (b, 0, 0))
    bs_h = pl.BlockSpec((1, ROW_BLK, HID), lambda b, r: (b, r, 0))
    bs_agg = pl.BlockSpec((1, 2, ROW_BLK, HID), lambda b, r: (b, 0, r, 0))
    return pl.pallas_call(
        functools.partial(_mlp_body, last=last),
        grid=grid,
        in_specs=[bs_small, bs_h, bs_agg, bs_w, bs_small, bs_w, bs_small],
        out_specs=bs_h,
        out_shape=jax.ShapeDtypeStruct((2, N_NODES, HID), jnp.float32),
    )(scale, h, agg, w1, b1, w2, b2)


def _heads_body(h_ref, m_ref,
                a1_ref, a1b_ref, a2_ref, a2b_ref, spp_ref,
                genw_ref, genb_ref, fpp_ref,
                g1a_ref, g1b_ref, g1bias_ref, g2_ref, g2b_ref,
                fw1a_ref, fw1b_ref, fb1_ref, fw2_ref, fb2_ref,
                clw_ref, clb_ref,
                logits_ref, ortho_ref):
    f32 = jnp.float32
    dn = (((0,), (0,)), ((), ()))

    def pool(m, hh):
        ssum = lax.dot_general(m, hh, dn, preferred_element_type=f32)
        cnt8 = lax.dot_general(m, jnp.ones((N_NODES, 8), f32), dn,
                               preferred_element_type=f32)
        cnt = cnt8[:, 0:1]
        return ssum / jnp.maximum(cnt, 1.0)

    sf = pool(m_ref[0], h_ref[0])
    ff = pool(m_ref[1], h_ref[1])

    z = jnp.maximum(jnp.dot(sf, a1_ref[...], preferred_element_type=f32)
                    + a1b_ref[...], 0.0)
    wl = jnp.dot(z, a2_ref[...], preferred_element_type=f32) + a2b_ref[...]
    wl = wl - jnp.max(wl, axis=-1, keepdims=True)
    we = jnp.exp(wl)
    w = we / jnp.sum(we, axis=-1, keepdims=True)
    sf = sf + jnp.dot(w, spp_ref[...], preferred_element_type=f32)

    dyn = jnp.dot(ff, genw_ref[...], preferred_element_type=f32) + genb_ref[...]
    static = jnp.sum(fpp_ref[...], axis=0, keepdims=True) / 5.0
    g = jnp.maximum(jnp.dot(ff, g1a_ref[...], preferred_element_type=f32)
                    + jnp.dot(ff, g1b_ref[...], preferred_element_type=f32)
                    + g1bias_ref[...], 0.0)
    z2 = jnp.dot(g, g2_ref[...], preferred_element_type=f32) + g2b_ref[...]
    gate = 1.0 / (1.0 + jnp.exp(-z2[:, 0:1]))
    ff = ff + gate * dyn + (1.0 - gate) * static

    n1 = jnp.sqrt(jnp.sum(sf * sf, axis=1, keepdims=True))
    n2 = jnp.sqrt(jnp.sum(ff * ff, axis=1, keepdims=True))
    f1 = sf / jnp.maximum(n1, 1e-12)
    f2 = ff / jnp.maximum(n2, 1e-12)
    cross = lax.dot_general(f1, f2, (((1,), (1,)), ((), ())),
                            preferred_element_type=f32)
    ortho_ref[...] = jnp.reshape(
        jnp.sum(jnp.abs(cross)) * (0.01 / (N_GRAPHS * N_GRAPHS)), (1, 1))

    fused = jnp.maximum(jnp.dot(sf, fw1a_ref[...], preferred_element_type=f32)
                        + jnp.dot(ff, fw1b_ref[...], preferred_element_type=f32)
                        + fb1_ref[...], 0.0)
    fused = jnp.dot(fused, fw2_ref[...], preferred_element_type=f32) + fb2_ref[...]
    logits_ref[...] = (jnp.dot(fused, clw_ref[...], preferred_element_type=f32)
                       + clb_ref[...])


def _heads_call(h, m, hp):
    return pl.pallas_call(
        _heads_body,
        out_shape=[jax.ShapeDtypeStruct((N_GRAPHS, HID), jnp.float32),
                   jax.ShapeDtypeStruct((1, 1), jnp.float32)],
    )(h, m, *hp)


def _pad_to(a, shape):
    pads = [(0, t - s) for s, t in zip(a.shape, shape)]
    return jnp.pad(a, pads)


def _prep_head_params(p):
    f32 = jnp.float32
    a1 = _pad_to(p["sp_A1"], (HID, HID))
    a1b = _pad_to(p["sp_a1"].reshape(1, -1), (1, HID))
    a2 = _pad_to(p["sp_A2"], (HID, HID))
    a2b = jnp.full((1, HID), -1e30, f32).at[0, :5].set(p["sp_a2"])
    spp = _pad_to(p["sp_prompts"], (HID, HID))
    genw = p["fp_gen_W"]
    genb = p["fp_gen_b"].reshape(1, HID)
    fpp = _pad_to(p["fp_prompts"], (8, HID))
    g1a = p["fp_G1"][:HID]
    g1b = p["fp_G1"][HID:]
    g1bias = p["fp_g1"].reshape(1, HID)
    g2 = _pad_to(p["fp_G2"], (HID, HID))
    g2b = _pad_to(p["fp_g2"].reshape(1, -1), (1, HID))
    fw1a = p["fu_W1"][:HID]
    fw1b = p["fu_W1"][HID:]
    fb1 = p["fu_b1"].reshape(1, HID)
    fw2 = p["fu_W2"]
    fb2 = p["fu_b2"].reshape(1, HID)
    clw = _pad_to(p["cl_W"], (HID, HID))
    clb = _pad_to(p["cl_b"].reshape(1, -1), (1, HID))
    return [a1, a1b, a2, a2b, spp, genw, genb, fpp, g1a, g1b, g1bias, g2, g2b,
            fw1a, fw1b, fb1, fw2, fb2, clw, clb]


def kernel(struct_x, struct_edge_index, struct_batch,
           func_x, func_edge_index, func_batch, params):
    edges_s = _prep_edges(struct_edge_index)
    edges_f = _prep_edges(func_edge_index)

    h = jnp.stack([struct_x, func_x])
    sl = params["struct_layers"]
    fl = params["func_layers"]
    for l in range(5):
        agg_s, agg_f = _seg_call(h, edges_s, edges_f)
        agg = jnp.stack([agg_s, agg_f])
        scale = jnp.stack([
            jnp.full((1, HID), 1.0 + sl[l]["eps"], jnp.float32),
            jnp.full((1, HID), 1.0 + fl[l]["eps"], jnp.float32),
        ])
        w1 = jnp.stack([sl[l]["W1"], fl[l]["W1"]])
        b1 = jnp.stack([sl[l]["b1"].reshape(1, HID), fl[l]["b1"].reshape(1, HID)])
        w2 = jnp.stack([sl[l]["W2"], fl[l]["W2"]])
        b2 = jnp.stack([sl[l]["b2"].reshape(1, HID), fl[l]["b2"].reshape(1, HID)])
        h = _mlp_call(scale, h, agg, w1, b1, w2, b2, last=(l == 4))

    gids = jnp.arange(N_GRAPHS, dtype=jnp.int32)
    m = jnp.stack([
        (struct_batch.astype(jnp.int32)[:, None] == gids[None, :]).astype(jnp.float32),
        (func_batch.astype(jnp.int32)[:, None] == gids[None, :]).astype(jnp.float32),
    ])
    logits_full, ortho = _heads_call(h, m, _prep_head_params(params))
    return (logits_full[:, :2], ortho.reshape(()))
